# baseline (speedup 1.0000x reference)
import jax
import jax.numpy as jnp
from jax import lax
from jax.experimental import pallas as pl
from jax.experimental.pallas import tpu as pltpu
from jax.experimental.pallas import tpu_sc as plsc

NUM_FIELDS = 26
FIELD_SIZE = 40000
BATCH = 16384
L = 16
NC = 2
NS = 16
NW = NC * NS
B_PER_W = BATCH // NW
EPW = B_PER_W * NUM_FIELDS
CHUNK = 128
NCHUNK = EPW // CHUNK


def _worker_id():
    return lax.axis_index("s") * NC + lax.axis_index("c")


def _body(x_hbm, table_hbm, bias_hbm, out_hbm, xb_v, idx_v, vals_v, out_v, bias_v, sem):
    wid = _worker_id()
    ebase = pl.multiple_of(wid * EPW, 8)
    pltpu.sync_copy(x_hbm.at[pl.ds(ebase, EPW)], xb_v)
    pltpu.sync_copy(bias_hbm, bias_v)

    iota = lax.iota(jnp.int32, L)

    def fire(r, carry):
        for k in range(CHUNK // L):
            c = r * (CHUNK // L) + k
            s = pl.ds(c * L, L)
            f = lax.rem(iota + c * L, NUM_FIELDS)
            idx_v[s] = xb_v[s] + f * FIELD_SIZE
        pltpu.async_copy(
            table_hbm.at[idx_v.at[pl.ds(r * CHUNK, CHUNK)]],
            vals_v.at[pl.ds(r * CHUNK, CHUNK)],
            sem,
        )
        return carry

    lax.fori_loop(0, NCHUNK, fire, 0)

    def drain(r, carry):
        pltpu.make_async_copy(
            table_hbm.at[pl.ds(0, CHUNK)],
            vals_v.at[pl.ds(r * CHUNK, CHUNK)],
            sem,
        ).wait()
        return carry

    lax.fori_loop(0, NCHUNK, drain, 0)

    bias_b = plsc.load_gather(bias_v, [iota * 0])
    iota26 = iota * NUM_FIELDS

    def red(c, carry):
        acc = bias_b
        cbase = c * (L * NUM_FIELDS)
        for f in range(NUM_FIELDS):
            acc = acc + plsc.load_gather(vals_v, [iota26 + (cbase + f)])
        out_v[pl.ds(c * L, L)] = acc
        return carry

    lax.fori_loop(0, B_PER_W // L, red, 0)

    obase = pl.multiple_of(wid * B_PER_W, 8)
    pltpu.sync_copy(out_v, out_hbm.at[pl.ds(obase, B_PER_W)])


@jax.jit
def _run(x_flat, table, bias):
    mesh = plsc.VectorSubcoreMesh(core_axis_name="c", subcore_axis_name="s")
    k = pl.kernel(
        _body,
        out_type=jax.ShapeDtypeStruct((BATCH,), jnp.float32),
        mesh=mesh,
        scratch_types=[
            pltpu.VMEM((EPW,), jnp.int32),
            pltpu.VMEM((EPW,), jnp.int32),
            pltpu.VMEM((EPW,), jnp.float32),
            pltpu.VMEM((B_PER_W,), jnp.float32),
            pltpu.VMEM((1,), jnp.float32),
            pltpu.SemaphoreType.DMA,
        ],
        name="features_linear_fused",
        compiler_params=pltpu.CompilerParams(needs_layout_passes=False),
    )
    return k(x_flat, table, bias)


def kernel(x, fc_weight, bias):
    x_flat = x.reshape(-1).astype(jnp.int32)
    table = fc_weight.reshape(-1)
    out = _run(x_flat, table, bias.astype(jnp.float32))
    return out.reshape(BATCH, 1)
